# baseline (speedup 1.0000x reference)
import functools

import jax
import jax.numpy as jnp
from jax import lax
from jax.experimental import pallas as pl
from jax.experimental.pallas import tpu as pltpu
from jax.experimental.pallas import tpu_sc as plsc

K = 512
B2 = 2 * K
LSTRIDE = B2 + 8
HWORDS = 16 * LSTRIDE
NC, NS = 2, 16
NW = NC * NS
ROWN = 100000
ROWP = 100352
NPAD = 4 * ROWP
PER_W = NPAD // NW
CHUNK = 3136
NVEC = CHUNK // 16
NCHUNK = PER_W // CHUNK
INV = 0x40004189
SCALE = 511.99

_mesh = plsc.VectorSubcoreMesh(
    core_axis_name="c", subcore_axis_name="s", num_cores=NC, num_subcores=NS
)


def _pack_body(p_ref, l_ref, o_ref):
    pi = lax.bitcast_convert_type(p_ref[:, 2, :], jnp.int32)
    lb = l_ref[...]
    pk = jnp.where(lb == 0, jnp.int32(INV), (pi & ~jnp.int32(3)) | lb)
    o_ref[...] = jnp.full((NPAD,), INV, jnp.int32)
    for r in range(4):
        o_ref[pl.ds(r * ROWP, ROWN)] = pk[r, :]


_pack_tc = pl.pallas_call(
    _pack_body,
    out_shape=jax.ShapeDtypeStruct((NPAD,), jnp.int32),
)


@functools.partial(
    pl.kernel,
    out_type=jax.ShapeDtypeStruct((NW, B2), jnp.int32),
    mesh=_mesh,
    scratch_types=[
        pltpu.VMEM((CHUNK,), jnp.int32),
        pltpu.VMEM((CHUNK,), jnp.int32),
        pltpu.VMEM((HWORDS,), jnp.int32),
        pltpu.VMEM((B2,), jnp.int32),
        pltpu.SemaphoreType.DMA,
        pltpu.SemaphoreType.DMA,
    ],
    compiler_params=pltpu.CompilerParams(needs_layout_passes=False),
)
def _hist(packed_hbm, t_hbm, buf0, buf1, hist, tloc, sem0, sem1):
    c = lax.axis_index("c")
    s = lax.axis_index("s")
    w = c * NS + s
    iot = lax.iota(jnp.int32, 16)
    lane_base = iot * LSTRIDE
    ones = jnp.ones((16,), jnp.int32)
    zeros = jnp.zeros((16,), jnp.int32)

    base = w * PER_W

    def _copy(ci, buf, sem):
        return pltpu.make_async_copy(
            packed_hbm.at[pl.ds(base + ci * CHUNK, CHUNK)], buf, sem
        )

    _copy(0, buf0, sem0).start()

    def _zero(i, carry):
        for u in range(8):
            hist[pl.ds(i * 128 + u * 16, 16)] = zeros
        return carry

    lax.fori_loop(0, HWORDS // 128, _zero, 0)

    def _bucket(pk):
        lb = pk & 3
        p = lax.bitcast_convert_type(pk, jnp.float32)
        q = jnp.where(lb == 2, 2.0 - p, p)
        bi = (q * jnp.float32(SCALE)).astype(jnp.int32)
        return bi + lane_base

    def _consume(buf):
        def _vec(v, carry2):
            pks = [buf[pl.ds(v * 64 + 16 * u, 16)] for u in range(4)]
            ixs = [_bucket(pk) for pk in pks]
            for ix in ixs:
                plsc.addupdate_scatter(hist, [ix], ones)
            return carry2

        lax.fori_loop(0, NVEC // 4, _vec, 0)

    def _pair(i, carry):
        _copy(2 * i + 1, buf1, sem1).start()
        _copy(2 * i, buf0, sem0).wait()
        _consume(buf0)

        @pl.when(2 * i + 2 < NCHUNK)
        def _():
            _copy(2 * i + 2, buf0, sem0).start()

        _copy(2 * i + 1, buf1, sem1).wait()
        _consume(buf1)
        return carry

    lax.fori_loop(0, NCHUNK // 2, _pair, 0)

    def _fold(g, carry):
        acc = hist[pl.ds(g * 16, 16)]
        for r in range(1, 16):
            acc = acc + hist[pl.ds(r * LSTRIDE + g * 16, 16)]
        tloc[pl.ds(g * 16, 16)] = acc
        return carry

    lax.fori_loop(0, B2 // 16, _fold, 0)

    pltpu.sync_copy(tloc, t_hbm.at[w])


def _scan_body(t_ref, o_ref):
    h = jnp.sum(t_ref[...].astype(jnp.float32), axis=0, keepdims=True)
    h0 = h[:, :K]
    h1 = h[:, K:]
    bi = lax.broadcasted_iota(jnp.int32, (K, K), 0)
    bj = lax.broadcasted_iota(jnp.int32, (K, K), 1)
    suf = (bi >= bj).astype(jnp.float32)
    f0 = jnp.dot(h0, suf, preferred_element_type=jnp.float32)
    f1 = jnp.dot(h1, suf, preferred_element_type=jnp.float32)
    g = jnp.sum(h1)
    den = g + f0
    j = 1.0 - (g - f1) / jnp.maximum(den, 1.0)
    j = jnp.where(den == 0.0, 0.0, j)
    col = lax.broadcasted_iota(jnp.int32, (1, K), 1)
    jlast = jnp.sum(jnp.where(col == 0, j, 0.0))
    o_ref[0, 0] = (jnp.sum(j) - 0.5 * jlast) * jnp.float32(1.0 / K)


_scan_tc = pl.pallas_call(
    _scan_body,
    out_shape=jax.ShapeDtypeStruct((1, 1), jnp.float32),
    out_specs=pl.BlockSpec(memory_space=pltpu.SMEM),
)


def kernel(probas, labels):
    packed = _pack_tc(probas, labels.astype(jnp.int32))
    t = _hist(packed)
    out = _scan_tc(t)
    return out[0, 0]


# speedup vs baseline: 1.0212x; 1.0212x over previous
import functools

import jax
import jax.numpy as jnp
from jax import lax
from jax.experimental import pallas as pl
from jax.experimental.pallas import tpu as pltpu
from jax.experimental.pallas import tpu_sc as plsc

K = 512
B2 = 2 * K
LSTRIDE = B2 + 8
HWORDS = 16 * LSTRIDE
NC, NS = 2, 16
NW = NC * NS
ROWN = 100000
ROWP = 100352
NPAD = 4 * ROWP
PER_W = NPAD // NW
CHUNK = 3136
NVEC = CHUNK // 16
NCHUNK = PER_W // CHUNK
INV = 0x40004189
SCALE = 511.99

_mesh = plsc.VectorSubcoreMesh(
    core_axis_name="c", subcore_axis_name="s", num_cores=NC, num_subcores=NS
)


ALN = (ROWN // 128) * 128


def _pack_body(p_ref, l_ref, o_ref):
    b = pl.program_id(0)
    pi = lax.bitcast_convert_type(p_ref[0, 2, :], jnp.int32)
    lb = l_ref[b, :]
    pk = jnp.where(lb == 0, jnp.int32(INV), (pi & ~jnp.int32(3)) | lb)
    tail = jnp.concatenate(
        [lax.slice(pk, (ALN,), (ROWN,)),
         jnp.full((ROWP - ROWN,), INV, jnp.int32)]
    )
    o_ref[pl.ds(0, ALN)] = lax.slice(pk, (0,), (ALN,))
    o_ref[pl.ds(ALN, ROWP - ALN)] = tail


_pack_tc = pl.pallas_call(
    _pack_body,
    grid=(4,),
    in_specs=[
        pl.BlockSpec((1, 3, ROWN), lambda b: (b, 0, 0)),
        pl.BlockSpec((4, ROWN), lambda b: (0, 0)),
    ],
    out_specs=pl.BlockSpec((ROWP,), lambda b: (b,)),
    out_shape=jax.ShapeDtypeStruct((NPAD,), jnp.int32),
)


@functools.partial(
    pl.kernel,
    out_type=jax.ShapeDtypeStruct((NW, B2), jnp.int32),
    mesh=_mesh,
    scratch_types=[
        pltpu.VMEM((CHUNK,), jnp.int32),
        pltpu.VMEM((CHUNK,), jnp.int32),
        pltpu.VMEM((HWORDS,), jnp.int32),
        pltpu.VMEM((B2,), jnp.int32),
        pltpu.SemaphoreType.DMA,
        pltpu.SemaphoreType.DMA,
    ],
    compiler_params=pltpu.CompilerParams(needs_layout_passes=False),
)
def _hist(packed_hbm, t_hbm, buf0, buf1, hist, tloc, sem0, sem1):
    c = lax.axis_index("c")
    s = lax.axis_index("s")
    w = c * NS + s
    iot = lax.iota(jnp.int32, 16)
    lane_base = iot * LSTRIDE
    ones = jnp.ones((16,), jnp.int32)
    zeros = jnp.zeros((16,), jnp.int32)

    base = w * PER_W

    def _copy(ci, buf, sem):
        return pltpu.make_async_copy(
            packed_hbm.at[pl.ds(base + ci * CHUNK, CHUNK)], buf, sem
        )

    _copy(0, buf0, sem0).start()

    def _zero(i, carry):
        for u in range(8):
            hist[pl.ds(i * 128 + u * 16, 16)] = zeros
        return carry

    lax.fori_loop(0, HWORDS // 128, _zero, 0)

    def _bucket(pk):
        lb = pk & 3
        p = lax.bitcast_convert_type(pk, jnp.float32)
        q = jnp.where(lb == 2, 2.0 - p, p)
        bi = (q * jnp.float32(SCALE)).astype(jnp.int32)
        return bi + lane_base

    def _consume(buf):
        def _vec(v, carry2):
            pks = [buf[pl.ds(v * 64 + 16 * u, 16)] for u in range(4)]
            ixs = [_bucket(pk) for pk in pks]
            for ix in ixs:
                plsc.addupdate_scatter(hist, [ix], ones)
            return carry2

        lax.fori_loop(0, NVEC // 4, _vec, 0)

    def _pair(i, carry):
        _copy(2 * i + 1, buf1, sem1).start()
        _copy(2 * i, buf0, sem0).wait()
        _consume(buf0)

        @pl.when(2 * i + 2 < NCHUNK)
        def _():
            _copy(2 * i + 2, buf0, sem0).start()

        _copy(2 * i + 1, buf1, sem1).wait()
        _consume(buf1)
        return carry

    lax.fori_loop(0, NCHUNK // 2, _pair, 0)

    def _fold(g, carry):
        acc = hist[pl.ds(g * 16, 16)]
        for r in range(1, 16):
            acc = acc + hist[pl.ds(r * LSTRIDE + g * 16, 16)]
        tloc[pl.ds(g * 16, 16)] = acc
        return carry

    lax.fori_loop(0, B2 // 16, _fold, 0)

    pltpu.sync_copy(tloc, t_hbm.at[w])


def _scan_body(t_ref, o_ref):
    h = jnp.sum(t_ref[...].astype(jnp.float32), axis=0, keepdims=True)
    h0 = h[:, :K]
    h1 = h[:, K:]
    bi = lax.broadcasted_iota(jnp.int32, (K, K), 0)
    bj = lax.broadcasted_iota(jnp.int32, (K, K), 1)
    suf = (bi >= bj).astype(jnp.float32)
    f0 = jnp.dot(h0, suf, preferred_element_type=jnp.float32)
    f1 = jnp.dot(h1, suf, preferred_element_type=jnp.float32)
    g = jnp.sum(h1)
    den = g + f0
    j = 1.0 - (g - f1) / jnp.maximum(den, 1.0)
    j = jnp.where(den == 0.0, 0.0, j)
    col = lax.broadcasted_iota(jnp.int32, (1, K), 1)
    jlast = jnp.sum(jnp.where(col == 0, j, 0.0))
    o_ref[0, 0] = (jnp.sum(j) - 0.5 * jlast) * jnp.float32(1.0 / K)


_scan_tc = pl.pallas_call(
    _scan_body,
    out_shape=jax.ShapeDtypeStruct((1, 1), jnp.float32),
    out_specs=pl.BlockSpec(memory_space=pltpu.SMEM),
)


def kernel(probas, labels):
    packed = _pack_tc(probas, labels.astype(jnp.int32))
    t = _hist(packed)
    out = _scan_tc(t)
    return out[0, 0]


# speedup vs baseline: 1.2286x; 1.2030x over previous
import functools

import jax
import jax.numpy as jnp
from jax import lax
from jax.experimental import pallas as pl
from jax.experimental.pallas import tpu as pltpu
from jax.experimental.pallas import tpu_sc as plsc

K = 512
B2 = 2 * K
LSTRIDE = B2 + 8
HWORDS = 16 * LSTRIDE
NC, NS = 2, 16
NW = NC * NS
PER_W = 12800
CHUNK = 1600
NVEC = CHUNK // 16
NCHUNK = PER_W // CHUNK
NPAD = NW * PER_W
INV = 0x40004189
SCALE = 511.99

_mesh = plsc.VectorSubcoreMesh(
    core_axis_name="c", subcore_axis_name="s", num_cores=NC, num_subcores=NS
)


@functools.partial(
    pl.kernel,
    out_type=jax.ShapeDtypeStruct((NW, B2), jnp.int32),
    mesh=_mesh,
    scratch_types=[
        pltpu.VMEM((CHUNK,), jnp.int32),
        pltpu.VMEM((CHUNK,), jnp.int32),
        pltpu.VMEM((HWORDS,), jnp.int32),
        pltpu.VMEM((B2,), jnp.int32),
        pltpu.SemaphoreType.DMA,
        pltpu.SemaphoreType.DMA,
    ],
    compiler_params=pltpu.CompilerParams(needs_layout_passes=False),
)
def _hist(packed_hbm, t_hbm, buf0, buf1, hist, tloc, sem0, sem1):
    c = lax.axis_index("c")
    s = lax.axis_index("s")
    w = c * NS + s
    iot = lax.iota(jnp.int32, 16)
    lane_base = iot * LSTRIDE
    ones = jnp.ones((16,), jnp.int32)
    zeros = jnp.zeros((16,), jnp.int32)

    base = w * PER_W

    def _copy(ci, buf, sem):
        return pltpu.make_async_copy(
            packed_hbm.at[pl.ds(base + ci * CHUNK, CHUNK)], buf, sem
        )

    _copy(0, buf0, sem0).start()

    def _zero(i, carry):
        for u in range(8):
            hist[pl.ds(i * 128 + u * 16, 16)] = zeros
        return carry

    lax.fori_loop(0, HWORDS // 128, _zero, 0)

    def _bucket(pk):
        lb = pk & 3
        p = lax.bitcast_convert_type(pk, jnp.float32)
        q = jnp.where(lb == 2, 2.0 - p, p)
        bi = (q * jnp.float32(SCALE)).astype(jnp.int32)
        return bi + lane_base

    def _consume(buf):
        def _vec(v, carry2):
            pks = [buf[pl.ds(v * 64 + 16 * u, 16)] for u in range(4)]
            ixs = [_bucket(pk) for pk in pks]
            for ix in ixs:
                plsc.addupdate_scatter(hist, [ix], ones)
            return carry2

        lax.fori_loop(0, NVEC // 4, _vec, 0)

    def _pair(i, carry):
        _copy(2 * i + 1, buf1, sem1).start()
        _copy(2 * i, buf0, sem0).wait()
        _consume(buf0)

        @pl.when(2 * i + 2 < NCHUNK)
        def _():
            _copy(2 * i + 2, buf0, sem0).start()

        _copy(2 * i + 1, buf1, sem1).wait()
        _consume(buf1)
        return carry

    lax.fori_loop(0, NCHUNK // 2, _pair, 0)

    def _fold(g, carry):
        acc = hist[pl.ds(g * 16, 16)]
        for r in range(1, 16):
            acc = acc + hist[pl.ds(r * LSTRIDE + g * 16, 16)]
        tloc[pl.ds(g * 16, 16)] = acc
        return carry

    lax.fori_loop(0, B2 // 16, _fold, 0)

    pltpu.sync_copy(tloc, t_hbm.at[w])


def _scan_body(t_ref, o_ref):
    h = jnp.sum(t_ref[...].astype(jnp.float32), axis=0, keepdims=True)
    h0 = h[:, :K]
    h1 = h[:, K:]
    bi = lax.broadcasted_iota(jnp.int32, (K, K), 0)
    bj = lax.broadcasted_iota(jnp.int32, (K, K), 1)
    suf = (bi >= bj).astype(jnp.float32)
    f0 = jnp.dot(h0, suf, preferred_element_type=jnp.float32)
    f1 = jnp.dot(h1, suf, preferred_element_type=jnp.float32)
    g = jnp.sum(h1)
    den = g + f0
    j = 1.0 - (g - f1) / jnp.maximum(den, 1.0)
    j = jnp.where(den == 0.0, 0.0, j)
    col = lax.broadcasted_iota(jnp.int32, (1, K), 1)
    jlast = jnp.sum(jnp.where(col == 0, j, 0.0))
    o_ref[0, 0] = (jnp.sum(j) - 0.5 * jlast) * jnp.float32(1.0 / K)


_scan_tc = pl.pallas_call(
    _scan_body,
    out_shape=jax.ShapeDtypeStruct((1, 1), jnp.float32),
    out_specs=pl.BlockSpec(memory_space=pltpu.SMEM),
)


def kernel(probas, labels):
    p = probas[:, 2, :].reshape(-1)
    lab = labels.reshape(-1).astype(jnp.int32)
    pi = lax.bitcast_convert_type(p, jnp.int32)
    packed = jnp.where(lab == 0, jnp.int32(INV), (pi & ~jnp.int32(3)) | lab)
    packed = jnp.concatenate(
        [packed, jnp.full((NPAD - packed.shape[0],), INV, jnp.int32)]
    )
    t = _hist(packed)
    out = _scan_tc(t)
    return out[0, 0]


# speedup vs baseline: 1.2453x; 1.0136x over previous
import functools

import jax
import jax.numpy as jnp
from jax import lax
from jax.experimental import pallas as pl
from jax.experimental.pallas import tpu as pltpu
from jax.experimental.pallas import tpu_sc as plsc

K = 256
B2 = 2 * K
LSTRIDE = B2 + 8
HWORDS = 16 * LSTRIDE
NC, NS = 2, 16
NW = NC * NS
PER_W = 12800
CHUNK = 1600
NVEC = CHUNK // 16
NCHUNK = PER_W // CHUNK
NPAD = NW * PER_W
INV = 0x40004189
SCALE = 255.99

_mesh = plsc.VectorSubcoreMesh(
    core_axis_name="c", subcore_axis_name="s", num_cores=NC, num_subcores=NS
)


@functools.partial(
    pl.kernel,
    out_type=jax.ShapeDtypeStruct((NW, B2), jnp.int32),
    mesh=_mesh,
    scratch_types=[
        pltpu.VMEM((CHUNK,), jnp.int32),
        pltpu.VMEM((CHUNK,), jnp.int32),
        pltpu.VMEM((HWORDS,), jnp.int32),
        pltpu.VMEM((B2,), jnp.int32),
        pltpu.SemaphoreType.DMA,
        pltpu.SemaphoreType.DMA,
    ],
    compiler_params=pltpu.CompilerParams(needs_layout_passes=False),
)
def _hist(packed_hbm, t_hbm, buf0, buf1, hist, tloc, sem0, sem1):
    c = lax.axis_index("c")
    s = lax.axis_index("s")
    w = c * NS + s
    iot = lax.iota(jnp.int32, 16)
    lane_base = iot * LSTRIDE
    ones = jnp.ones((16,), jnp.int32)
    zeros = jnp.zeros((16,), jnp.int32)

    base = w * PER_W

    def _copy(ci, buf, sem):
        return pltpu.make_async_copy(
            packed_hbm.at[pl.ds(base + ci * CHUNK, CHUNK)], buf, sem
        )

    _copy(0, buf0, sem0).start()

    def _zero(i, carry):
        for u in range(8):
            hist[pl.ds(i * 128 + u * 16, 16)] = zeros
        return carry

    lax.fori_loop(0, HWORDS // 128, _zero, 0)

    def _bucket(pk):
        lb = pk & 3
        p = lax.bitcast_convert_type(pk, jnp.float32)
        q = jnp.where(lb == 2, 2.0 - p, p)
        bi = (q * jnp.float32(SCALE)).astype(jnp.int32)
        return bi + lane_base

    def _consume(buf):
        def _vec(v, carry2):
            pks = [buf[pl.ds(v * 64 + 16 * u, 16)] for u in range(4)]
            ixs = [_bucket(pk) for pk in pks]
            for ix in ixs:
                plsc.addupdate_scatter(hist, [ix], ones)
            return carry2

        lax.fori_loop(0, NVEC // 4, _vec, 0)

    def _pair(i, carry):
        _copy(2 * i + 1, buf1, sem1).start()
        _copy(2 * i, buf0, sem0).wait()
        _consume(buf0)

        @pl.when(2 * i + 2 < NCHUNK)
        def _():
            _copy(2 * i + 2, buf0, sem0).start()

        _copy(2 * i + 1, buf1, sem1).wait()
        _consume(buf1)
        return carry

    lax.fori_loop(0, NCHUNK // 2, _pair, 0)

    def _fold(g, carry):
        acc = hist[pl.ds(g * 16, 16)]
        for r in range(1, 16):
            acc = acc + hist[pl.ds(r * LSTRIDE + g * 16, 16)]
        tloc[pl.ds(g * 16, 16)] = acc
        return carry

    lax.fori_loop(0, B2 // 16, _fold, 0)

    pltpu.sync_copy(tloc, t_hbm.at[w])


def _scan_body(t_ref, o_ref):
    h = jnp.sum(t_ref[...].astype(jnp.float32), axis=0, keepdims=True)
    h0 = h[:, :K]
    h1 = h[:, K:]
    bi = lax.broadcasted_iota(jnp.int32, (K, K), 0)
    bj = lax.broadcasted_iota(jnp.int32, (K, K), 1)
    suf = (bi >= bj).astype(jnp.float32)
    f0 = jnp.dot(h0, suf, preferred_element_type=jnp.float32)
    f1 = jnp.dot(h1, suf, preferred_element_type=jnp.float32)
    g = jnp.sum(h1)
    den = g + f0
    j = 1.0 - (g - f1) / jnp.maximum(den, 1.0)
    j = jnp.where(den == 0.0, 0.0, j)
    col = lax.broadcasted_iota(jnp.int32, (1, K), 1)
    jlast = jnp.sum(jnp.where(col == 0, j, 0.0))
    o_ref[0, 0] = (jnp.sum(j) - 0.5 * jlast) * jnp.float32(1.0 / K)


_scan_tc = pl.pallas_call(
    _scan_body,
    out_shape=jax.ShapeDtypeStruct((1, 1), jnp.float32),
    out_specs=pl.BlockSpec(memory_space=pltpu.SMEM),
)


def kernel(probas, labels):
    p = probas[:, 2, :].reshape(-1)
    lab = labels.reshape(-1).astype(jnp.int32)
    pi = lax.bitcast_convert_type(p, jnp.int32)
    packed = jnp.where(lab == 0, jnp.int32(INV), (pi & ~jnp.int32(3)) | lab)
    packed = jnp.concatenate(
        [packed, jnp.full((NPAD - packed.shape[0],), INV, jnp.int32)]
    )
    t = _hist(packed)
    out = _scan_tc(t)
    return out[0, 0]


# speedup vs baseline: 1.3105x; 1.0524x over previous
import functools

import jax
import jax.numpy as jnp
from jax import lax
from jax.experimental import pallas as pl
from jax.experimental.pallas import tpu as pltpu
from jax.experimental.pallas import tpu_sc as plsc

K = 256
B2 = 2 * K
LSTRIDE = B2 + 8
HWORDS = 16 * LSTRIDE
NC, NS = 2, 16
NW = NC * NS
PER_W = 12800
CHUNK = 3200
NVEC = CHUNK // 16
NCHUNK = PER_W // CHUNK
NPAD = NW * PER_W
INV = 0x40004189
SCALE = 255.99

_mesh = plsc.VectorSubcoreMesh(
    core_axis_name="c", subcore_axis_name="s", num_cores=NC, num_subcores=NS
)


@functools.partial(
    pl.kernel,
    out_type=jax.ShapeDtypeStruct((NW, B2), jnp.int32),
    mesh=_mesh,
    scratch_types=[
        pltpu.VMEM((CHUNK,), jnp.int32),
        pltpu.VMEM((CHUNK,), jnp.int32),
        pltpu.VMEM((HWORDS,), jnp.int32),
        pltpu.VMEM((B2,), jnp.int32),
        pltpu.SemaphoreType.DMA,
        pltpu.SemaphoreType.DMA,
    ],
    compiler_params=pltpu.CompilerParams(needs_layout_passes=False),
)
def _hist(packed_hbm, t_hbm, buf0, buf1, hist, tloc, sem0, sem1):
    c = lax.axis_index("c")
    s = lax.axis_index("s")
    w = c * NS + s
    iot = lax.iota(jnp.int32, 16)
    lane_base = iot * LSTRIDE
    ones = jnp.ones((16,), jnp.int32)
    zeros = jnp.zeros((16,), jnp.int32)

    base = w * PER_W

    def _copy(ci, buf, sem):
        return pltpu.make_async_copy(
            packed_hbm.at[pl.ds(base + ci * CHUNK, CHUNK)], buf, sem
        )

    _copy(0, buf0, sem0).start()

    def _zero(i, carry):
        for u in range(8):
            hist[pl.ds(i * 128 + u * 16, 16)] = zeros
        return carry

    lax.fori_loop(0, HWORDS // 128, _zero, 0)

    def _bucket(pk):
        lb = pk & 3
        p = lax.bitcast_convert_type(pk, jnp.float32)
        q = jnp.where(lb == 2, 2.0 - p, p)
        bi = (q * jnp.float32(SCALE)).astype(jnp.int32)
        return bi + lane_base

    def _consume(buf):
        def _vec(v, carry2):
            pks = [buf[pl.ds(v * 128 + 16 * u, 16)] for u in range(8)]
            ixs = [_bucket(pk) for pk in pks]
            for ix in ixs:
                plsc.addupdate_scatter(hist, [ix], ones)
            return carry2

        lax.fori_loop(0, NVEC // 8, _vec, 0)

    def _pair(i, carry):
        _copy(2 * i + 1, buf1, sem1).start()
        _copy(2 * i, buf0, sem0).wait()
        _consume(buf0)

        @pl.when(2 * i + 2 < NCHUNK)
        def _():
            _copy(2 * i + 2, buf0, sem0).start()

        _copy(2 * i + 1, buf1, sem1).wait()
        _consume(buf1)
        return carry

    lax.fori_loop(0, NCHUNK // 2, _pair, 0)

    def _fold(g, carry):
        acc = hist[pl.ds(g * 16, 16)]
        for r in range(1, 16):
            acc = acc + hist[pl.ds(r * LSTRIDE + g * 16, 16)]
        tloc[pl.ds(g * 16, 16)] = acc
        return carry

    lax.fori_loop(0, B2 // 16, _fold, 0)

    pltpu.sync_copy(tloc, t_hbm.at[w])


def _scan_body(t_ref, o_ref):
    h = jnp.sum(t_ref[...].astype(jnp.float32), axis=0, keepdims=True)
    h0 = h[:, :K]
    h1 = h[:, K:]
    bi = lax.broadcasted_iota(jnp.int32, (K, K), 0)
    bj = lax.broadcasted_iota(jnp.int32, (K, K), 1)
    suf = (bi >= bj).astype(jnp.float32)
    f0 = jnp.dot(h0, suf, preferred_element_type=jnp.float32)
    f1 = jnp.dot(h1, suf, preferred_element_type=jnp.float32)
    g = jnp.sum(h1)
    den = g + f0
    j = 1.0 - (g - f1) / jnp.maximum(den, 1.0)
    j = jnp.where(den == 0.0, 0.0, j)
    col = lax.broadcasted_iota(jnp.int32, (1, K), 1)
    jlast = jnp.sum(jnp.where(col == 0, j, 0.0))
    o_ref[0, 0] = (jnp.sum(j) - 0.5 * jlast) * jnp.float32(1.0 / K)


_scan_tc = pl.pallas_call(
    _scan_body,
    out_shape=jax.ShapeDtypeStruct((1, 1), jnp.float32),
    out_specs=pl.BlockSpec(memory_space=pltpu.SMEM),
)


def kernel(probas, labels):
    p = probas[:, 2, :].reshape(-1)
    lab = labels.reshape(-1).astype(jnp.int32)
    pi = lax.bitcast_convert_type(p, jnp.int32)
    packed = jnp.where(lab == 0, jnp.int32(INV), (pi & ~jnp.int32(3)) | lab)
    packed = jnp.concatenate(
        [packed, jnp.full((NPAD - packed.shape[0],), INV, jnp.int32)]
    )
    t = _hist(packed)
    out = _scan_tc(t)
    return out[0, 0]


# speedup vs baseline: 1.3571x; 1.0356x over previous
import functools

import jax
import jax.numpy as jnp
from jax import lax
from jax.experimental import pallas as pl
from jax.experimental.pallas import tpu as pltpu
from jax.experimental.pallas import tpu_sc as plsc

K = 256
B2 = 2 * K
LSTRIDE = B2 + 8
HWORDS = 16 * LSTRIDE
NC, NS = 2, 16
NW = NC * NS
PER_W = 12800
CHUNK = 3200
NVEC = CHUNK // 16
NCHUNK = PER_W // CHUNK
N = 400000
INV = 0x40004189
SCALE = 255.99

_mesh = plsc.VectorSubcoreMesh(
    core_axis_name="c", subcore_axis_name="s", num_cores=NC, num_subcores=NS
)


@functools.partial(
    pl.kernel,
    out_type=jax.ShapeDtypeStruct((NW, B2), jnp.int32),
    mesh=_mesh,
    scratch_types=[
        pltpu.VMEM((CHUNK,), jnp.int32),
        pltpu.VMEM((CHUNK,), jnp.int32),
        pltpu.VMEM((HWORDS,), jnp.int32),
        pltpu.VMEM((B2,), jnp.int32),
        pltpu.SemaphoreType.DMA,
        pltpu.SemaphoreType.DMA,
    ],
    compiler_params=pltpu.CompilerParams(needs_layout_passes=False),
)
def _hist(packed_hbm, t_hbm, buf0, buf1, hist, tloc, sem0, sem1):
    c = lax.axis_index("c")
    s = lax.axis_index("s")
    w = c * NS + s
    iot = lax.iota(jnp.int32, 16)
    lane_base = iot * LSTRIDE
    ones = jnp.ones((16,), jnp.int32)
    zeros = jnp.zeros((16,), jnp.int32)

    base = w * PER_W
    nch = jnp.where(w == NW - 1, (N - (NW - 1) * PER_W) // CHUNK,
                    PER_W // CHUNK)
    npair = nch // 2

    def _copy(ci, buf, sem):
        return pltpu.make_async_copy(
            packed_hbm.at[pl.ds(base + ci * CHUNK, CHUNK)], buf, sem
        )

    _copy(0, buf0, sem0).start()

    def _zero(i, carry):
        for u in range(8):
            hist[pl.ds(i * 128 + u * 16, 16)] = zeros
        return carry

    lax.fori_loop(0, HWORDS // 128, _zero, 0)

    def _bucket(pk):
        lb = pk & 3
        p = lax.bitcast_convert_type(pk, jnp.float32)
        q = jnp.where(lb == 2, 2.0 - p, p)
        bi = (q * jnp.float32(SCALE)).astype(jnp.int32)
        return bi + lane_base

    def _consume(buf):
        def _vec(v, carry2):
            pks = [buf[pl.ds(v * 128 + 16 * u, 16)] for u in range(8)]
            ixs = [_bucket(pk) for pk in pks]
            for ix in ixs:
                plsc.addupdate_scatter(hist, [ix], ones)
            return carry2

        lax.fori_loop(0, NVEC // 8, _vec, 0)

    def _pair(i, carry):
        _copy(2 * i + 1, buf1, sem1).start()
        _copy(2 * i, buf0, sem0).wait()
        _consume(buf0)

        @pl.when(2 * i + 2 < nch)
        def _():
            _copy(2 * i + 2, buf0, sem0).start()

        _copy(2 * i + 1, buf1, sem1).wait()
        _consume(buf1)
        return carry

    lax.fori_loop(0, npair, _pair, 0)

    @pl.when(nch == 1)
    def _():
        _copy(0, buf0, sem0).wait()
        _consume(buf0)

    def _fold(g, carry):
        acc = hist[pl.ds(g * 16, 16)]
        for r in range(1, 16):
            acc = acc + hist[pl.ds(r * LSTRIDE + g * 16, 16)]
        tloc[pl.ds(g * 16, 16)] = acc
        return carry

    lax.fori_loop(0, B2 // 16, _fold, 0)

    pltpu.sync_copy(tloc, t_hbm.at[w])


def _scan_body(t_ref, o_ref):
    h = jnp.sum(t_ref[...].astype(jnp.float32), axis=0, keepdims=True)
    h0 = h[:, :K]
    h1 = h[:, K:]
    bi = lax.broadcasted_iota(jnp.int32, (K, K), 0)
    bj = lax.broadcasted_iota(jnp.int32, (K, K), 1)
    suf = (bi >= bj).astype(jnp.float32)
    f0 = jnp.dot(h0, suf, preferred_element_type=jnp.float32)
    f1 = jnp.dot(h1, suf, preferred_element_type=jnp.float32)
    g = jnp.sum(h1)
    den = g + f0
    j = 1.0 - (g - f1) / jnp.maximum(den, 1.0)
    j = jnp.where(den == 0.0, 0.0, j)
    col = lax.broadcasted_iota(jnp.int32, (1, K), 1)
    jlast = jnp.sum(jnp.where(col == 0, j, 0.0))
    o_ref[0, 0] = (jnp.sum(j) - 0.5 * jlast) * jnp.float32(1.0 / K)


_scan_tc = pl.pallas_call(
    _scan_body,
    out_shape=jax.ShapeDtypeStruct((1, 1), jnp.float32),
    out_specs=pl.BlockSpec(memory_space=pltpu.SMEM),
)


def kernel(probas, labels):
    lab2 = labels.astype(jnp.int32)
    pi2 = lax.bitcast_convert_type(probas[:, 2, :], jnp.int32)
    pk2 = jnp.where(lab2 == 0, jnp.int32(INV), (pi2 & ~jnp.int32(3)) | lab2)
    t = _hist(pk2.reshape(-1))
    out = _scan_tc(t)
    return out[0, 0]
